# baseline (speedup 1.0000x reference)
import functools

import jax
import jax.numpy as jnp
import numpy as np
from jax import lax
from jax.experimental import pallas as pl
from jax.experimental.pallas import tpu as pltpu
from jax.experimental.pallas import tpu_sc as plsc

GX = GY = GZ = 128
NUM_RAYS = 4096
NUM_SAMPLES = 64
VOXEL_DIM = 28
PAD = 32

Y_0_0 = 0.5 * np.sqrt(1.0 / np.pi)
HALF_SQRT_3_BY_PI = 0.5 * np.sqrt(3.0 / np.pi)
QUARTER_SQRT_5_BY_PI = 0.25 * np.sqrt(5.0 / np.pi)
HALF_SQRT_15_BY_PI = 0.5 * np.sqrt(15.0 / np.pi)
QUARTER_SQRT_15_BY_PI = 0.25 * np.sqrt(15.0 / np.pi)

NW = 32
RPT = NUM_RAYS // NW
NG = NUM_SAMPLES // 16
NVOX = GX * GY * GZ
CHUNK = (GY * GZ) // NW
IDX_PER_RAY = 4 * 5 * NUM_SAMPLES
WPITCH = 29

_SC_PARAMS = pltpu.CompilerParams(
    use_tc_tiling_on_sc=False, needs_layout_passes=False
)
_MESH = dict(core_axis_name="c", subcore_axis_name="s", num_cores=2,
             num_subcores=16)


def _basis_tc(viewing_angles):
    def body(va_ref, out_ref):
        th = va_ref[:, 0:1]
        ph = va_ref[:, 1:2]
        st, ct = jnp.sin(th), jnp.cos(th)
        sp, cp = jnp.sin(ph), jnp.cos(ph)
        cols = [
            jnp.full_like(th, Y_0_0),
            HALF_SQRT_3_BY_PI * st * sp,
            HALF_SQRT_3_BY_PI * ct,
            HALF_SQRT_3_BY_PI * st * cp,
            HALF_SQRT_15_BY_PI * st * cp * st * sp,
            HALF_SQRT_15_BY_PI * st * sp * ct,
            QUARTER_SQRT_5_BY_PI * (3.0 * ct * ct - 1.0),
            HALF_SQRT_15_BY_PI * st * cp * ct,
            QUARTER_SQRT_15_BY_PI * ((st * cp) ** 2 - (st * sp) ** 2),
        ]
        li = lax.broadcasted_iota(jnp.int32, (NUM_RAYS, 16), 1)
        acc = jnp.zeros((NUM_RAYS, 16), jnp.float32)
        for k, c in enumerate(cols):
            acc += jnp.where(li == k, c, 0.0)
        out_ref[:, :] = acc

    return pl.pallas_call(
        body,
        out_shape=jax.ShapeDtypeStruct((NUM_RAYS, 16), jnp.float32),
    )(viewing_angles)


def _sc_convert(src2d):
    mesh = plsc.VectorSubcoreMesh(**_MESH)

    @functools.partial(
        pl.kernel,
        out_type=jax.ShapeDtypeStruct((NVOX * WPITCH,), jnp.float32),
        mesh=mesh,
        compiler_params=_SC_PARAMS,
        scratch_types=[
            pltpu.VMEM((2, PAD, CHUNK + 1), jnp.float32),
            pltpu.VMEM((2, CHUNK * WPITCH + 16), jnp.float32),
            pltpu.SemaphoreType.DMA,
            pltpu.SemaphoreType.DMA,
            pltpu.SemaphoreType.DMA,
            pltpu.SemaphoreType.DMA,
        ],
    )
    def k(src, dst, in_v, out_v, si0, si1, so0, so1):
        wid = lax.axis_index("s") * 2 + lax.axis_index("c")
        col0 = wid * CHUNK
        sins = (si0, si1)
        souts = (so0, so1)
        lane = lax.iota(jnp.int32, 16)

        def in_copy(x, b):
            return pltpu.make_async_copy(
                src.at[pl.ds(x * VOXEL_DIM, VOXEL_DIM), pl.ds(col0, CHUNK)],
                in_v.at[b, pl.ds(0, VOXEL_DIM), pl.ds(0, CHUNK)],
                sins[b],
            )

        def out_copy(x, b):
            off = (x * (GY * GZ) + col0) * WPITCH
            return pltpu.make_async_copy(
                out_v.at[b, pl.ds(0, CHUNK * WPITCH)],
                dst.at[pl.ds(off, CHUNK * WPITCH)],
                souts[b],
            )

        in_copy(0, 0).start()

        def body(i, c):
            for b in (0, 1):
                x = 2 * i + b

                @pl.when(x + 1 < GX)
                def _():
                    in_copy(x + 1, 1 - b).start()

                in_copy(x, b).wait()

                @pl.when(x >= 2)
                def _():
                    out_copy(x - 2, b).wait()

                in2 = in_v.at[b]
                lane_hi = lane + 16

                def sgrp(t, cc):
                    s16 = 16 * t
                    csp = jnp.full((16,), s16, jnp.int32)
                    bvec = csp * WPITCH + lane
                    outb = out_v.at[b]
                    for u in range(16):
                        col = csp + u
                        v0 = plsc.load_gather(in2, [lane, col])
                        v1 = plsc.load_gather(in2, [lane_hi, col])
                        idx0 = bvec + u * WPITCH
                        plsc.store_scatter(outb, [idx0], v0)
                        plsc.store_scatter(outb, [idx0 + 16], v1)
                    return cc

                lax.fori_loop(0, CHUNK // 16, sgrp, jnp.int32(0))
                out_copy(x, b).start()
            return c

        lax.fori_loop(0, GX // 2, body, jnp.int32(0))
        out_copy(GX - 2, 0).wait()
        out_copy(GX - 1, 1).wait()

    return k(src2d)


def _sc_render(tab16, positions, distances, basis):
    mesh = plsc.VectorSubcoreMesh(**_MESH)

    @functools.partial(
        pl.kernel,
        out_type=jax.ShapeDtypeStruct((NUM_RAYS, 3), jnp.float32),
        mesh=mesh,
        compiler_params=_SC_PARAMS,
        scratch_types=[
            pltpu.VMEM((RPT, NUM_SAMPLES * 3), jnp.float32),
            pltpu.VMEM((RPT, NUM_SAMPLES), jnp.float32),
            pltpu.VMEM((RPT, 16), jnp.float32),
            pltpu.VMEM((2, 512), jnp.float32),
            pltpu.VMEM((2, 256), jnp.int32),
            pltpu.VMEM((2, IDX_PER_RAY), jnp.int32),
            pltpu.VMEM((2, IDX_PER_RAY, 16), jnp.float32),
            pltpu.VMEM((RPT, 3), jnp.float32),
            pltpu.SemaphoreType.DMA,
            pltpu.SemaphoreType.DMA,
        ],
    )
    def k(grid, pos, dist, bas, out, pos_v, dist_v, basis_v, wbuf, obuf,
          idxb, rows_v, out_v, sem0, sem1):
        wid = lax.axis_index("s") * 2 + lax.axis_index("c")
        ray0 = wid * RPT
        pltpu.sync_copy(pos.at[pl.ds(ray0, RPT)], pos_v)
        pltpu.sync_copy(dist.at[pl.ds(ray0, RPT)], dist_v)
        pltpu.sync_copy(bas.at[pl.ds(ray0, RPT)], basis_v)

        lane = lax.iota(jnp.int32, 16)
        sems = (sem0, sem1)

        def build(ray, b):
            rayv = jnp.full((16,), ray, jnp.int32)

            def grp(g, c):
                s0 = 16 * g
                sv = (lane + s0) * 3
                x = plsc.load_gather(pos_v, [rayv, sv])
                y = plsc.load_gather(pos_v, [rayv, sv + 1])
                z = plsc.load_gather(pos_v, [rayv, sv + 2])
                xi = x.astype(jnp.int32)
                yi = y.astype(jnp.int32)
                zi = z.astype(jnp.int32)
                xd = x - xi.astype(jnp.float32)
                yd = y - yi.astype(jnp.float32)
                zd = z - zi.astype(jnp.float32)
                vid = xi * (GY * GZ) + yi * GZ + zi
                wx = (1.0 - xd, xd)
                wy = (1.0 - yd, yd)
                wz = (1.0 - zd, zd)
                for p in range(4):
                    dx, dy = (p >> 1) & 1, p & 1
                    vp = vid + (dx * (GY * GZ) + dy * GZ)
                    w28 = vp * WPITCH
                    r16 = lax.shift_right_logical(w28, 4)
                    al = w28 & 15
                    slot = p * 64 + s0 + lane
                    slot5 = slot * 5
                    for q in range(5):
                        plsc.store_scatter(idxb.at[b], [slot5 + q], r16 + q)
                    obuf[b, pl.ds(p * 64 + s0, 16)] = slot * 80 + al
                    for dz in range(2):
                        woff = (p * 2 + dz) * 64 + s0
                        wbuf[b, pl.ds(woff, 16)] = wx[dx] * wy[dy] * wz[dz]
                return c

            lax.fori_loop(0, NG, grp, jnp.int32(0))

        def fire(b):
            for i in range(IDX_PER_RAY // 128):
                pltpu.async_copy(
                    grid.at[idxb.at[b, pl.ds(i * 128, 128)]],
                    rows_v.at[b, pl.ds(i * 128, 128)],
                    sems[b],
                )

        def drain(b):
            for i in range(IDX_PER_RAY // 128):
                pltpu.make_async_copy(
                    grid.at[idxb.at[b, pl.ds(i * 128, 128)]],
                    rows_v.at[b, pl.ds(i * 128, 128)],
                    sems[b],
                ).wait()

        def blend(ray, b):
            rayv = jnp.full((16,), ray, jnp.int32)
            bk = [
                plsc.load_gather(basis_v, [rayv, jnp.full((16,), kk, jnp.int32)])
                for kk in range(9)
            ]
            rowsb = rows_v.at[b]

            def grp(g, carry4):
                racc, gacc, bacc, csum_c = carry4
                s0 = 16 * g
                base = [obuf[b, pl.ds(p * 64 + s0, 16)] for p in range(4)]
                wv = [wbuf[b, pl.ds(r * 64 + s0, 16)] for r in range(8)]

                def chan(j):
                    acc = None
                    for p in range(4):
                        for dz in range(2):
                            t = base[p] + (WPITCH * dz + j)
                            row = lax.shift_right_logical(t, 4)
                            col = t & 15
                            v = plsc.load_gather(rowsb, [row, col])
                            term = wv[p * 2 + dz] * v
                            acc = term if acc is None else acc + term
                    return acc

                sig = chan(0)
                cols = []
                for c in range(3):
                    col = bk[0] * chan(1 + 9 * c)
                    for kk in range(1, 9):
                        col += bk[kk] * chan(1 + 9 * c + kk)
                    cols.append(col)
                d_g = dist_v[ray, pl.ds(s0, 16)]
                att = jnp.exp(-sig * d_g)
                csum = plsc.cumsum(att) + csum_c
                w = csum * (1.0 - att)
                wm = jnp.where(sig != 0.0, w, 0.0)
                return (
                    racc + jnp.sum(wm * cols[0]),
                    gacc + jnp.sum(wm * cols[1]),
                    bacc + jnp.sum(wm * cols[2]),
                    csum_c + jnp.sum(att),
                )

            z = jnp.float32(0.0)
            racc, gacc, bacc, _ = lax.fori_loop(0, NG, grp, (z, z, z, z))
            rgbv = jnp.where(lane == 0, racc, jnp.where(lane == 1, gacc, bacc))
            plsc.store_scatter(out_v, [rayv, lane], rgbv, mask=lane < 3)

        build(jnp.int32(0), 0)
        fire(0)

        def body(i, c):
            r0 = 2 * i
            build(r0 + 1, 1)
            fire(1)
            drain(0)
            blend(r0, 0)

            @pl.when(i < (RPT // 2 - 1))
            def _():
                build(r0 + 2, 0)
                fire(0)

            drain(1)
            blend(r0 + 1, 1)
            return c

        lax.fori_loop(0, RPT // 2, body, jnp.int32(0))
        pltpu.sync_copy(out_v, out.at[pl.ds(ray0, RPT)])

    return k(tab16, positions, distances, basis)


def kernel(positions, distances, viewing_angles, voxel_grid):
    basis = _basis_tc(viewing_angles)
    src2d = voxel_grid.transpose(0, 3, 1, 2).reshape(GX * VOXEL_DIM, GY * GZ)
    dense = _sc_convert(src2d)
    tab16 = dense.reshape(NVOX * WPITCH // 16, 16)
    pos2d = positions.reshape(NUM_RAYS, NUM_SAMPLES * 3)
    return _sc_render(tab16, pos2d, distances, basis)


# speedup vs baseline: 1.1466x; 1.1466x over previous
import functools

import jax
import jax.numpy as jnp
import numpy as np
from jax import lax
from jax.experimental import pallas as pl
from jax.experimental.pallas import tpu as pltpu
from jax.experimental.pallas import tpu_sc as plsc

GX = GY = GZ = 128
NUM_RAYS = 4096
NUM_SAMPLES = 64
VOXEL_DIM = 28
PAD = 32

Y_0_0 = 0.5 * np.sqrt(1.0 / np.pi)
HALF_SQRT_3_BY_PI = 0.5 * np.sqrt(3.0 / np.pi)
QUARTER_SQRT_5_BY_PI = 0.25 * np.sqrt(5.0 / np.pi)
HALF_SQRT_15_BY_PI = 0.5 * np.sqrt(15.0 / np.pi)
QUARTER_SQRT_15_BY_PI = 0.25 * np.sqrt(15.0 / np.pi)

NW = 32
RPT = NUM_RAYS // NW
NG = NUM_SAMPLES // 16
NVOX = GX * GY * GZ
CHUNK = (GY * GZ) // NW
IDX_PER_RAY = 4 * 5 * NUM_SAMPLES
WPITCH = 29

_SC_PARAMS = pltpu.CompilerParams(
    use_tc_tiling_on_sc=False, needs_layout_passes=False
)
_MESH = dict(core_axis_name="c", subcore_axis_name="s", num_cores=2,
             num_subcores=16)


def _basis_tc(viewing_angles):
    def body(va_ref, out_ref):
        th = va_ref[:, 0:1]
        ph = va_ref[:, 1:2]
        st, ct = jnp.sin(th), jnp.cos(th)
        sp, cp = jnp.sin(ph), jnp.cos(ph)
        cols = [
            jnp.full_like(th, Y_0_0),
            HALF_SQRT_3_BY_PI * st * sp,
            HALF_SQRT_3_BY_PI * ct,
            HALF_SQRT_3_BY_PI * st * cp,
            HALF_SQRT_15_BY_PI * st * cp * st * sp,
            HALF_SQRT_15_BY_PI * st * sp * ct,
            QUARTER_SQRT_5_BY_PI * (3.0 * ct * ct - 1.0),
            HALF_SQRT_15_BY_PI * st * cp * ct,
            QUARTER_SQRT_15_BY_PI * ((st * cp) ** 2 - (st * sp) ** 2),
        ]
        li = lax.broadcasted_iota(jnp.int32, (NUM_RAYS, 16), 1)
        acc = jnp.zeros((NUM_RAYS, 16), jnp.float32)
        for k, c in enumerate(cols):
            acc += jnp.where(li == k, c, 0.0)
        out_ref[:, :] = acc

    return pl.pallas_call(
        body,
        out_shape=jax.ShapeDtypeStruct((NUM_RAYS, 16), jnp.float32),
    )(viewing_angles)


def _sc_convert(src2d):
    mesh = plsc.VectorSubcoreMesh(**_MESH)

    @functools.partial(
        pl.kernel,
        out_type=jax.ShapeDtypeStruct((NVOX * WPITCH,), jnp.float32),
        mesh=mesh,
        compiler_params=_SC_PARAMS,
        scratch_types=[
            pltpu.VMEM((2, PAD, CHUNK + 1), jnp.float32),
            pltpu.VMEM((2, CHUNK * WPITCH + 16), jnp.float32),
            pltpu.SemaphoreType.DMA,
            pltpu.SemaphoreType.DMA,
            pltpu.SemaphoreType.DMA,
            pltpu.SemaphoreType.DMA,
        ],
    )
    def k(src, dst, in_v, out_v, si0, si1, so0, so1):
        wid = lax.axis_index("s") * 2 + lax.axis_index("c")
        col0 = wid * CHUNK
        sins = (si0, si1)
        souts = (so0, so1)
        lane = lax.iota(jnp.int32, 16)

        def in_copy(x, b):
            return pltpu.make_async_copy(
                src.at[pl.ds(x * VOXEL_DIM, VOXEL_DIM), pl.ds(col0, CHUNK)],
                in_v.at[b, pl.ds(0, VOXEL_DIM), pl.ds(0, CHUNK)],
                sins[b],
            )

        def out_copy(x, b):
            off = (x * (GY * GZ) + col0) * WPITCH
            return pltpu.make_async_copy(
                out_v.at[b, pl.ds(0, CHUNK * WPITCH)],
                dst.at[pl.ds(off, CHUNK * WPITCH)],
                souts[b],
            )

        in_copy(0, 0).start()

        def body(i, c):
            for b in (0, 1):
                x = 2 * i + b

                @pl.when(x + 1 < GX)
                def _():
                    in_copy(x + 1, 1 - b).start()

                in_copy(x, b).wait()

                @pl.when(x >= 2)
                def _():
                    out_copy(x - 2, b).wait()

                in2 = in_v.at[b]
                lane_hi = lane + 16

                def sgrp(t, cc):
                    s64 = 64 * t
                    for q in range(4):
                        s16 = s64 + 16 * q
                        csp = jnp.full((16,), s16, jnp.int32)
                        base = s16 * WPITCH
                        for u in range(16):
                            col = csp + u
                            v0 = plsc.load_gather(in2, [lane, col])
                            v1 = plsc.load_gather(in2, [lane_hi, col])
                            out_v[b, pl.ds(base + u * WPITCH, 16)] = v0
                            out_v[b, pl.ds(base + u * WPITCH + 16, 16)] = v1
                    return cc

                lax.fori_loop(0, CHUNK // 64, sgrp, jnp.int32(0))
                out_copy(x, b).start()
            return c

        lax.fori_loop(0, GX // 2, body, jnp.int32(0))
        out_copy(GX - 2, 0).wait()
        out_copy(GX - 1, 1).wait()

    return k(src2d)


def _sc_render(tab16, positions, distances, basis):
    mesh = plsc.VectorSubcoreMesh(**_MESH)

    @functools.partial(
        pl.kernel,
        out_type=jax.ShapeDtypeStruct((NUM_RAYS, 3), jnp.float32),
        mesh=mesh,
        compiler_params=_SC_PARAMS,
        scratch_types=[
            pltpu.VMEM((RPT, NUM_SAMPLES * 3), jnp.float32),
            pltpu.VMEM((RPT, NUM_SAMPLES), jnp.float32),
            pltpu.VMEM((RPT, 16), jnp.float32),
            pltpu.VMEM((2, 512), jnp.float32),
            pltpu.VMEM((2, 256), jnp.int32),
            pltpu.VMEM((2, IDX_PER_RAY), jnp.int32),
            pltpu.VMEM((2, IDX_PER_RAY, 16), jnp.float32),
            pltpu.VMEM((RPT, 3), jnp.float32),
            pltpu.SemaphoreType.DMA,
            pltpu.SemaphoreType.DMA,
        ],
    )
    def k(grid, pos, dist, bas, out, pos_v, dist_v, basis_v, wbuf, obuf,
          idxb, rows_v, out_v, sem0, sem1):
        wid = lax.axis_index("s") * 2 + lax.axis_index("c")
        ray0 = wid * RPT
        pltpu.sync_copy(pos.at[pl.ds(ray0, RPT)], pos_v)
        pltpu.sync_copy(dist.at[pl.ds(ray0, RPT)], dist_v)
        pltpu.sync_copy(bas.at[pl.ds(ray0, RPT)], basis_v)

        lane = lax.iota(jnp.int32, 16)
        sems = (sem0, sem1)

        def build(ray, b):
            rayv = jnp.full((16,), ray, jnp.int32)

            def grp(g, c):
                s0 = 16 * g
                sv = (lane + s0) * 3
                x = plsc.load_gather(pos_v, [rayv, sv])
                y = plsc.load_gather(pos_v, [rayv, sv + 1])
                z = plsc.load_gather(pos_v, [rayv, sv + 2])
                xi = x.astype(jnp.int32)
                yi = y.astype(jnp.int32)
                zi = z.astype(jnp.int32)
                xd = x - xi.astype(jnp.float32)
                yd = y - yi.astype(jnp.float32)
                zd = z - zi.astype(jnp.float32)
                vid = xi * (GY * GZ) + yi * GZ + zi
                wx = (1.0 - xd, xd)
                wy = (1.0 - yd, yd)
                wz = (1.0 - zd, zd)
                for p in range(4):
                    dx, dy = (p >> 1) & 1, p & 1
                    vp = vid + (dx * (GY * GZ) + dy * GZ)
                    w28 = vp * WPITCH
                    r16 = lax.shift_right_logical(w28, 4)
                    al = w28 & 15
                    slot = p * 64 + s0 + lane
                    slot5 = slot * 5
                    for q in range(5):
                        plsc.store_scatter(idxb.at[b], [slot5 + q], r16 + q)
                    obuf[b, pl.ds(p * 64 + s0, 16)] = slot * 80 + al
                    for dz in range(2):
                        woff = (p * 2 + dz) * 64 + s0
                        wbuf[b, pl.ds(woff, 16)] = wx[dx] * wy[dy] * wz[dz]
                return c

            lax.fori_loop(0, NG, grp, jnp.int32(0))

        def fire(b):
            for i in range(IDX_PER_RAY // 128):
                pltpu.async_copy(
                    grid.at[idxb.at[b, pl.ds(i * 128, 128)]],
                    rows_v.at[b, pl.ds(i * 128, 128)],
                    sems[b],
                )

        def drain(b):
            for i in range(IDX_PER_RAY // 128):
                pltpu.make_async_copy(
                    grid.at[idxb.at[b, pl.ds(i * 128, 128)]],
                    rows_v.at[b, pl.ds(i * 128, 128)],
                    sems[b],
                ).wait()

        def blend(ray, b):
            rayv = jnp.full((16,), ray, jnp.int32)
            bk = [
                plsc.load_gather(basis_v, [rayv, jnp.full((16,), kk, jnp.int32)])
                for kk in range(9)
            ]
            rowsb = rows_v.at[b]

            def grp(g, carry4):
                racc, gacc, bacc, csum_c = carry4
                s0 = 16 * g
                base = [obuf[b, pl.ds(p * 64 + s0, 16)] for p in range(4)]
                wv = [wbuf[b, pl.ds(r * 64 + s0, 16)] for r in range(8)]

                def chan(j):
                    acc = None
                    for p in range(4):
                        for dz in range(2):
                            t = base[p] + (WPITCH * dz + j)
                            row = lax.shift_right_logical(t, 4)
                            col = t & 15
                            v = plsc.load_gather(rowsb, [row, col])
                            term = wv[p * 2 + dz] * v
                            acc = term if acc is None else acc + term
                    return acc

                sig = chan(0)
                cols = []
                for c in range(3):
                    col = bk[0] * chan(1 + 9 * c)
                    for kk in range(1, 9):
                        col += bk[kk] * chan(1 + 9 * c + kk)
                    cols.append(col)
                d_g = dist_v[ray, pl.ds(s0, 16)]
                att = jnp.exp(-sig * d_g)
                csum = plsc.cumsum(att) + csum_c
                w = csum * (1.0 - att)
                wm = jnp.where(sig != 0.0, w, 0.0)
                return (
                    racc + jnp.sum(wm * cols[0]),
                    gacc + jnp.sum(wm * cols[1]),
                    bacc + jnp.sum(wm * cols[2]),
                    csum_c + jnp.sum(att),
                )

            z = jnp.float32(0.0)
            racc, gacc, bacc, _ = lax.fori_loop(0, NG, grp, (z, z, z, z))
            rgbv = jnp.where(lane == 0, racc, jnp.where(lane == 1, gacc, bacc))
            plsc.store_scatter(out_v, [rayv, lane], rgbv, mask=lane < 3)

        build(jnp.int32(0), 0)
        fire(0)

        def body(i, c):
            r0 = 2 * i
            build(r0 + 1, 1)
            fire(1)
            drain(0)
            blend(r0, 0)

            @pl.when(i < (RPT // 2 - 1))
            def _():
                build(r0 + 2, 0)
                fire(0)

            drain(1)
            blend(r0 + 1, 1)
            return c

        lax.fori_loop(0, RPT // 2, body, jnp.int32(0))
        pltpu.sync_copy(out_v, out.at[pl.ds(ray0, RPT)])

    return k(tab16, positions, distances, basis)


def kernel(positions, distances, viewing_angles, voxel_grid):
    basis = _basis_tc(viewing_angles)
    src2d = voxel_grid.transpose(0, 3, 1, 2).reshape(GX * VOXEL_DIM, GY * GZ)
    dense = _sc_convert(src2d)
    tab16 = dense.reshape(NVOX * WPITCH // 16, 16)
    pos2d = positions.reshape(NUM_RAYS, NUM_SAMPLES * 3)
    return _sc_render(tab16, pos2d, distances, basis)


# speedup vs baseline: 1.1555x; 1.0077x over previous
import functools

import jax
import jax.numpy as jnp
import numpy as np
from jax import lax
from jax.experimental import pallas as pl
from jax.experimental.pallas import tpu as pltpu
from jax.experimental.pallas import tpu_sc as plsc

GX = GY = GZ = 128
NUM_RAYS = 4096
NUM_SAMPLES = 64
VOXEL_DIM = 28
PAD = 32

Y_0_0 = 0.5 * np.sqrt(1.0 / np.pi)
HALF_SQRT_3_BY_PI = 0.5 * np.sqrt(3.0 / np.pi)
QUARTER_SQRT_5_BY_PI = 0.25 * np.sqrt(5.0 / np.pi)
HALF_SQRT_15_BY_PI = 0.5 * np.sqrt(15.0 / np.pi)
QUARTER_SQRT_15_BY_PI = 0.25 * np.sqrt(15.0 / np.pi)

NW = 32
RPT = NUM_RAYS // NW
NG = NUM_SAMPLES // 16
NVOX = GX * GY * GZ
CHUNK = (GY * GZ) // NW
IDX_PER_RAY = 4 * 5 * NUM_SAMPLES
WPITCH = 29

_SC_PARAMS = pltpu.CompilerParams(
    use_tc_tiling_on_sc=False, needs_layout_passes=False
)
_MESH = dict(core_axis_name="c", subcore_axis_name="s", num_cores=2,
             num_subcores=16)


def _basis_tc(viewing_angles):
    def body(va_ref, out_ref):
        th = va_ref[:, 0:1]
        ph = va_ref[:, 1:2]
        st, ct = jnp.sin(th), jnp.cos(th)
        sp, cp = jnp.sin(ph), jnp.cos(ph)
        cols = [
            jnp.full_like(th, Y_0_0),
            HALF_SQRT_3_BY_PI * st * sp,
            HALF_SQRT_3_BY_PI * ct,
            HALF_SQRT_3_BY_PI * st * cp,
            HALF_SQRT_15_BY_PI * st * cp * st * sp,
            HALF_SQRT_15_BY_PI * st * sp * ct,
            QUARTER_SQRT_5_BY_PI * (3.0 * ct * ct - 1.0),
            HALF_SQRT_15_BY_PI * st * cp * ct,
            QUARTER_SQRT_15_BY_PI * ((st * cp) ** 2 - (st * sp) ** 2),
        ]
        li = lax.broadcasted_iota(jnp.int32, (NUM_RAYS, 16), 1)
        acc = jnp.zeros((NUM_RAYS, 16), jnp.float32)
        for k, c in enumerate(cols):
            acc += jnp.where(li == k, c, 0.0)
        out_ref[:, :] = acc

    return pl.pallas_call(
        body,
        out_shape=jax.ShapeDtypeStruct((NUM_RAYS, 16), jnp.float32),
    )(viewing_angles)


def _sc_convert(src2d):
    mesh = plsc.VectorSubcoreMesh(**_MESH)

    @functools.partial(
        pl.kernel,
        out_type=jax.ShapeDtypeStruct((NVOX * WPITCH,), jnp.float32),
        mesh=mesh,
        compiler_params=_SC_PARAMS,
        scratch_types=[
            pltpu.VMEM((2, PAD, CHUNK + 1), jnp.float32),
            pltpu.VMEM((2, CHUNK * WPITCH + 16), jnp.float32),
            pltpu.SemaphoreType.DMA,
            pltpu.SemaphoreType.DMA,
            pltpu.SemaphoreType.DMA,
            pltpu.SemaphoreType.DMA,
        ],
    )
    def k(src, dst, in_v, out_v, si0, si1, so0, so1):
        wid = lax.axis_index("s") * 2 + lax.axis_index("c")
        col0 = wid * CHUNK
        sins = (si0, si1)
        souts = (so0, so1)
        lane = lax.iota(jnp.int32, 16)

        def in_copy(x, b):
            return pltpu.make_async_copy(
                src.at[pl.ds(x * VOXEL_DIM, VOXEL_DIM), pl.ds(col0, CHUNK)],
                in_v.at[b, pl.ds(0, VOXEL_DIM), pl.ds(0, CHUNK)],
                sins[b],
            )

        def out_copy(x, b):
            off = (x * (GY * GZ) + col0) * WPITCH
            return pltpu.make_async_copy(
                out_v.at[b, pl.ds(0, CHUNK * WPITCH)],
                dst.at[pl.ds(off, CHUNK * WPITCH)],
                souts[b],
            )

        in_copy(0, 0).start()

        def body(i, c):
            for b in (0, 1):
                x = 2 * i + b

                @pl.when(x + 1 < GX)
                def _():
                    in_copy(x + 1, 1 - b).start()

                in_copy(x, b).wait()

                @pl.when(x >= 2)
                def _():
                    out_copy(x - 2, b).wait()

                in2 = in_v.at[b]
                lane_hi = lane + 16

                def sgrp(t, cc):
                    s16 = 16 * t
                    csp = jnp.full((16,), s16, jnp.int32)
                    base = s16 * WPITCH
                    for u in range(16):
                        col = csp + u
                        v0 = plsc.load_gather(in2, [lane, col])
                        v1 = plsc.load_gather(in2, [lane_hi, col])
                        out_v[b, pl.ds(base + u * WPITCH, 16)] = v0
                        out_v[b, pl.ds(base + u * WPITCH + 16, 16)] = v1
                    return cc

                lax.fori_loop(0, CHUNK // 16, sgrp, jnp.int32(0))
                out_copy(x, b).start()
            return c

        lax.fori_loop(0, GX // 2, body, jnp.int32(0))
        out_copy(GX - 2, 0).wait()
        out_copy(GX - 1, 1).wait()

    return k(src2d)


def _sc_render(tab16, positions, distances, basis):
    mesh = plsc.VectorSubcoreMesh(**_MESH)

    @functools.partial(
        pl.kernel,
        out_type=jax.ShapeDtypeStruct((NUM_RAYS, 3), jnp.float32),
        mesh=mesh,
        compiler_params=_SC_PARAMS,
        scratch_types=[
            pltpu.VMEM((RPT, NUM_SAMPLES * 3), jnp.float32),
            pltpu.VMEM((RPT, NUM_SAMPLES), jnp.float32),
            pltpu.VMEM((RPT, 16), jnp.float32),
            pltpu.VMEM((2, 512), jnp.float32),
            pltpu.VMEM((2, 256), jnp.int32),
            pltpu.VMEM((2, IDX_PER_RAY), jnp.int32),
            pltpu.VMEM((2, IDX_PER_RAY, 16), jnp.float32),
            pltpu.VMEM((RPT, 3), jnp.float32),
            pltpu.SemaphoreType.DMA,
            pltpu.SemaphoreType.DMA,
        ],
    )
    def k(grid, pos, dist, bas, out, pos_v, dist_v, basis_v, wbuf, obuf,
          idxb, rows_v, out_v, sem0, sem1):
        wid = lax.axis_index("s") * 2 + lax.axis_index("c")
        ray0 = wid * RPT
        pltpu.sync_copy(pos.at[pl.ds(ray0, RPT)], pos_v)
        pltpu.sync_copy(dist.at[pl.ds(ray0, RPT)], dist_v)
        pltpu.sync_copy(bas.at[pl.ds(ray0, RPT)], basis_v)

        lane = lax.iota(jnp.int32, 16)
        sems = (sem0, sem1)

        def build(ray, b):
            rayv = jnp.full((16,), ray, jnp.int32)

            def grp(g, c):
                s0 = 16 * g
                sv = (lane + s0) * 3
                x = plsc.load_gather(pos_v, [rayv, sv])
                y = plsc.load_gather(pos_v, [rayv, sv + 1])
                z = plsc.load_gather(pos_v, [rayv, sv + 2])
                xi = x.astype(jnp.int32)
                yi = y.astype(jnp.int32)
                zi = z.astype(jnp.int32)
                xd = x - xi.astype(jnp.float32)
                yd = y - yi.astype(jnp.float32)
                zd = z - zi.astype(jnp.float32)
                vid = xi * (GY * GZ) + yi * GZ + zi
                wx = (1.0 - xd, xd)
                wy = (1.0 - yd, yd)
                wz = (1.0 - zd, zd)
                for p in range(4):
                    dx, dy = (p >> 1) & 1, p & 1
                    vp = vid + (dx * (GY * GZ) + dy * GZ)
                    w28 = vp * WPITCH
                    r16 = lax.shift_right_logical(w28, 4)
                    al = w28 & 15
                    slot = p * 64 + s0 + lane
                    slot5 = slot * 5
                    for q in range(5):
                        plsc.store_scatter(idxb.at[b], [slot5 + q], r16 + q)
                    obuf[b, pl.ds(p * 64 + s0, 16)] = slot * 80 + al
                    for dz in range(2):
                        woff = (p * 2 + dz) * 64 + s0
                        wbuf[b, pl.ds(woff, 16)] = wx[dx] * wy[dy] * wz[dz]
                return c

            lax.fori_loop(0, NG, grp, jnp.int32(0))

        def fire(b):
            for i in range(IDX_PER_RAY // 128):
                pltpu.async_copy(
                    grid.at[idxb.at[b, pl.ds(i * 128, 128)]],
                    rows_v.at[b, pl.ds(i * 128, 128)],
                    sems[b],
                )

        def drain(b):
            for i in range(IDX_PER_RAY // 128):
                pltpu.make_async_copy(
                    grid.at[idxb.at[b, pl.ds(i * 128, 128)]],
                    rows_v.at[b, pl.ds(i * 128, 128)],
                    sems[b],
                ).wait()

        def blend(ray, b):
            rayv = jnp.full((16,), ray, jnp.int32)
            bk = [
                plsc.load_gather(basis_v, [rayv, jnp.full((16,), kk, jnp.int32)])
                for kk in range(9)
            ]
            rowsb = rows_v.at[b]

            def grp(g, carry4):
                racc, gacc, bacc, csum_c = carry4
                s0 = 16 * g
                base = [obuf[b, pl.ds(p * 64 + s0, 16)] for p in range(4)]
                wv = [wbuf[b, pl.ds(r * 64 + s0, 16)] for r in range(8)]

                def chan(j):
                    acc = None
                    for p in range(4):
                        for dz in range(2):
                            t = base[p] + (WPITCH * dz + j)
                            row = lax.shift_right_logical(t, 4)
                            col = t & 15
                            v = plsc.load_gather(rowsb, [row, col])
                            term = wv[p * 2 + dz] * v
                            acc = term if acc is None else acc + term
                    return acc

                sig = chan(0)
                cols = []
                for c in range(3):
                    col = bk[0] * chan(1 + 9 * c)
                    for kk in range(1, 9):
                        col += bk[kk] * chan(1 + 9 * c + kk)
                    cols.append(col)
                d_g = dist_v[ray, pl.ds(s0, 16)]
                att = jnp.exp(-sig * d_g)
                csum = plsc.cumsum(att) + csum_c
                w = csum * (1.0 - att)
                wm = jnp.where(sig != 0.0, w, 0.0)
                return (
                    racc + jnp.sum(wm * cols[0]),
                    gacc + jnp.sum(wm * cols[1]),
                    bacc + jnp.sum(wm * cols[2]),
                    csum_c + jnp.sum(att),
                )

            z = jnp.float32(0.0)
            racc, gacc, bacc, _ = lax.fori_loop(0, NG, grp, (z, z, z, z))
            rgbv = jnp.where(lane == 0, racc, jnp.where(lane == 1, gacc, bacc))
            plsc.store_scatter(out_v, [rayv, lane], rgbv, mask=lane < 3)

        build(jnp.int32(0), 0)
        fire(0)

        def body(i, c):
            r0 = 2 * i
            build(r0 + 1, 1)
            fire(1)
            drain(0)
            blend(r0, 0)

            @pl.when(i < (RPT // 2 - 1))
            def _():
                build(r0 + 2, 0)
                fire(0)

            drain(1)
            blend(r0 + 1, 1)
            return c

        lax.fori_loop(0, RPT // 2, body, jnp.int32(0))
        pltpu.sync_copy(out_v, out.at[pl.ds(ray0, RPT)])

    return k(tab16, positions, distances, basis)


def kernel(positions, distances, viewing_angles, voxel_grid):
    basis = _basis_tc(viewing_angles)
    src2d = voxel_grid.transpose(0, 3, 1, 2).reshape(GX * VOXEL_DIM, GY * GZ)
    dense = _sc_convert(src2d)
    tab16 = dense.reshape(NVOX * WPITCH // 16, 16)
    pos2d = positions.reshape(NUM_RAYS, NUM_SAMPLES * 3)
    return _sc_render(tab16, pos2d, distances, basis)
